# Initial kernel scaffold; baseline (speedup 1.0000x reference)
import jax
import jax.numpy as jnp
from jax.experimental import pallas as pl


def kernel(x, logits):
    raise NotImplementedError("write your pallas kernel here")


# speedup vs baseline: 1.3568x; 1.3568x over previous
import functools

import jax
import jax.numpy as jnp
from jax import lax
from jax.experimental import pallas as pl
from jax.experimental.pallas import tpu as pltpu
from jax.experimental.pallas import tpu_sc as plsc

_VOCAB = 1000
_D = 1000
_B = 1024
_H = 20
_N = _B * _H
_NC = 2
_NS = 16
_NW = _NC * _NS
_BPW = _N // _NW
_CH = 40
_NCHUNK = _BPW // _CH


def _gather_body(table_hbm, idx_hbm, out_hbm, idx_v, buf0, buf1,
                 sem_i0, sem_i1, sem_o0, sem_o1):
    wid = lax.axis_index("s") * _NC + lax.axis_index("c")
    base = wid * _BPW
    pltpu.sync_copy(idx_hbm.at[pl.ds(base, _BPW)], idx_v)

    bufs = (buf0, buf1)
    sin = (sem_i0, sem_i1)
    sout = (sem_o0, sem_o1)

    def start_gather(k):
        return pltpu.async_copy(
            table_hbm.at[idx_v.at[pl.ds(k * _CH, _CH)]], bufs[k % 2], sin[k % 2])

    def start_out(k):
        return pltpu.async_copy(
            bufs[k % 2], out_hbm.at[pl.ds(base + k * _CH, _CH)], sout[k % 2])

    in_fly = {0: start_gather(0)}
    out_fly = {}
    for k in range(_NCHUNK):
        nxt = k + 1
        if nxt < _NCHUNK:
            if nxt - 2 in out_fly:
                out_fly.pop(nxt - 2).wait()
            in_fly[nxt] = start_gather(nxt)
        in_fly.pop(k).wait()
        out_fly[k] = start_out(k)
    out_fly.pop(_NCHUNK - 2).wait()
    out_fly.pop(_NCHUNK - 1).wait()


@functools.partial(
    pl.kernel,
    mesh=plsc.VectorSubcoreMesh(core_axis_name="c", subcore_axis_name="s"),
    out_type=jax.ShapeDtypeStruct((_N, _D), jnp.float32),
    compiler_params=pltpu.CompilerParams(use_tc_tiling_on_sc=False),
    scratch_types=[
        pltpu.VMEM((_BPW,), jnp.int32),
        pltpu.VMEM((_CH, _D), jnp.float32),
        pltpu.VMEM((_CH, _D), jnp.float32),
        pltpu.SemaphoreType.DMA,
        pltpu.SemaphoreType.DMA,
        pltpu.SemaphoreType.DMA,
        pltpu.SemaphoreType.DMA,
    ],
)
def _sc_gather(table_hbm, idx_hbm, out_hbm, idx_v, buf0, buf1,
               sem_i0, sem_i1, sem_o0, sem_o1):
    _gather_body(table_hbm, idx_hbm, out_hbm, idx_v, buf0, buf1,
                 sem_i0, sem_i1, sem_o0, sem_o1)


def kernel(x, logits):
    idx = x.reshape(_N).astype(jnp.int32)
    out = _sc_gather(logits, idx)
    return out.reshape(_B, _H, _D)
